# baseline (speedup 1.0000x reference)
import functools

import jax
import jax.numpy as jnp
from jax import lax
from jax.experimental import pallas as pl
from jax.experimental.pallas import tpu as pltpu
from jax.experimental.pallas import tpu_sc as plsc

B, N, M = 4096, 2048, 2048
W = N + M
NV = W // 16
NC, NS = 2, 16
NWORK = NC * NS
RPW = B // NWORK
NDIG = 256
NPASS = 4


def _sc_body(x_hbm, y_hbm, xp_hbm, yp_hbm, out_hbm,
             key_a, val_a, key_b, val_b, hist, off, loss_buf):
    wid = lax.axis_index("s") * NC + lax.axis_index("c")
    base = wid * RPW
    lanes = lax.iota(jnp.int32, 16)
    ones_i = jnp.full((16,), 1, jnp.int32)
    zeros_i = jnp.zeros((16,), jnp.int32)

    def row_body(r, _):
        row = base + r
        pltpu.sync_copy(xp_hbm.at[row], key_a.at[pl.ds(0, N)])
        pltpu.sync_copy(yp_hbm.at[row], key_a.at[pl.ds(N, M)])
        pltpu.sync_copy(x_hbm.at[row], val_a.at[pl.ds(0, N)])
        pltpu.sync_copy(y_hbm.at[row], val_a.at[pl.ds(N, M)])

        def sum_half(i, acc, ref, base_off):
            return acc + ref[pl.ds(base_off + i * 16, 16)]

        sx_v = lax.fori_loop(0, N // 16, functools.partial(sum_half, ref=val_a, base_off=0),
                             jnp.zeros((16,), jnp.float32))
        sy_v = lax.fori_loop(0, M // 16, functools.partial(sum_half, ref=val_a, base_off=N),
                             jnp.zeros((16,), jnp.float32))
        ones_f = jnp.ones((16,), jnp.float32)
        inv_sx = ones_f / (ones_f * jnp.sum(sx_v))
        neg_inv_sy = (-ones_f) / (ones_f * jnp.sum(sy_v))

        def scale_body(i, s):
            sl = pl.ds(i * 16, 16)
            val_a[sl] = val_a[sl] * s
            return s

        lax.fori_loop(0, N // 16, scale_body, inv_sx)

        def scale_body2(i, s):
            sl = pl.ds(N + i * 16, 16)
            val_a[sl] = val_a[sl] * s
            return s

        lax.fori_loop(0, M // 16, scale_body2, neg_inv_sy)

        for p in range(NPASS):
            shift = 8 * p
            src_k, src_v = (key_a, val_a) if p % 2 == 0 else (key_b, val_b)
            dst_k, dst_v = (key_b, val_b) if p % 2 == 0 else (key_a, val_a)

            def zero_body(i, _):
                hist[pl.ds(i * 16, 16)] = zeros_i
                return 0

            lax.fori_loop(0, NV, zero_body, 0)

            def hist_body(i, _):
                k = src_k[pl.ds(i * 16, 16)]
                ki = plsc.bitcast(k, jnp.int32)
                d = lax.shift_right_logical(ki, shift) & (NDIG - 1)
                addr = (d << 4) | lanes
                plsc.addupdate_scatter(hist, [addr], ones_i)
                return 0

            lax.fori_loop(0, NV, hist_body, 0)

            def off_body(d, carry):
                hv = hist[pl.ds(d * 16, 16)]
                incl = plsc.cumsum(hv)
                excl = incl - hv
                plsc.store_scatter(off, [(lanes << 8) | d], carry + excl)
                return carry + jnp.sum(hv)

            lax.fori_loop(0, NDIG, off_body, jnp.int32(0))

            def perm_body(i, _):
                sl = pl.ds(i * 16, 16)
                k = src_k[sl]
                v = src_v[sl]
                ki = plsc.bitcast(k, jnp.int32)
                d = lax.shift_right_logical(ki, shift) & (NDIG - 1)
                taddr = (lanes << 8) | d
                g = plsc.load_gather(off, [taddr])
                plsc.addupdate_scatter(off, [taddr], ones_i)
                a = ((g & (NV - 1)) << 4) | lax.shift_right_logical(g, 8)
                plsc.store_scatter(dst_k, [a], k)
                plsc.store_scatter(dst_v, [a], v)
                return 0

            lax.fori_loop(0, NV, perm_body, 0)

        def tot_body(i, acc):
            return acc + val_a[pl.ds(i * 16, 16)]

        t = lax.fori_loop(0, NV, tot_body, jnp.zeros((16,), jnp.float32))
        lane_base = plsc.cumsum(t) - t

        def loss_body(i, carry):
            acc, lsum = carry
            sl = pl.ds(i * 16, 16)
            k = key_a[sl]
            nk = key_a[pl.ds((i + 1) * 16, 16)]
            acc = acc + val_a[sl]
            d_here = lane_base + acc
            lsum = lsum + (nk - k) * jnp.abs(d_here)
            return acc, lsum

        acc, lsum = lax.fori_loop(
            0, NV - 1, loss_body,
            (jnp.zeros((16,), jnp.float32), jnp.zeros((16,), jnp.float32)))
        k_end = key_a[pl.ds((NV - 1) * 16, 16)]
        acc = acc + val_a[pl.ds((NV - 1) * 16, 16)]
        d_end = lane_base + acc
        nxt_start = plsc.load_gather(key_a, [jnp.minimum(lanes + 1, 15)])
        seam = jnp.where(lanes < 15, (nxt_start - k_end) * jnp.abs(d_end), 0.0)
        loss = jnp.sum(lsum) + jnp.sum(seam)
        plsc.store_scatter(loss_buf, [jnp.full((16,), 1, jnp.int32) * r],
                           jnp.zeros((16,), jnp.float32) + loss,
                           mask=lanes == 0)
        return 0

    lax.fori_loop(0, RPW, row_body, 0)
    pltpu.sync_copy(loss_buf, out_hbm.at[pl.ds(base, RPW)])


@jax.jit
def kernel(x, y, x_pos, y_pos):
    mesh = plsc.VectorSubcoreMesh(core_axis_name="c", subcore_axis_name="s",
                                  num_cores=NC, num_subcores=NS)
    f = pl.kernel(
        _sc_body,
        out_type=jax.ShapeDtypeStruct((B,), jnp.float32),
        mesh=mesh,
        scratch_types=[
            pltpu.VMEM((W,), jnp.float32),
            pltpu.VMEM((W,), jnp.float32),
            pltpu.VMEM((W,), jnp.float32),
            pltpu.VMEM((W,), jnp.float32),
            pltpu.VMEM((W,), jnp.int32),
            pltpu.VMEM((W,), jnp.int32),
            pltpu.VMEM((RPW,), jnp.float32),
        ],
        compiler_params=pltpu.CompilerParams(needs_layout_passes=False),
    )
    return f(x, y, x_pos, y_pos)


# speedup vs baseline: 1.9165x; 1.9165x over previous
import jax
import jax.numpy as jnp
from jax import lax
from jax.experimental import pallas as pl
from jax.experimental.pallas import tpu as pltpu
from jax.experimental.pallas import tpu_sc as plsc

B, N, M = 4096, 2048, 2048
W = N + M
NV = W // 16
NVX = N // 16
NC, NS = 2, 16
NWORK = NC * NS
RPW = B // NWORK
NDIG = 256
NPASS = 4
UNROLL = 8


def _sc_body(x_hbm, y_hbm, xp_hbm, yp_hbm, out_hbm,
             key_a, val_a, key_b, val_b, key_s, val_s, hist, off, loss_buf,
             dma_sem):
    wid = lax.axis_index("s") * NC + lax.axis_index("c")
    base = wid * RPW
    lanes = lax.iota(jnp.int32, 16)
    ones_i = jnp.full((16,), 1, jnp.int32)
    zeros_i = jnp.zeros((16,), jnp.int32)
    zeros_f = jnp.zeros((16,), jnp.float32)
    ones_f = jnp.ones((16,), jnp.float32)

    def zero_body(i, _):
        hist[pl.ds(i * 16, 16)] = zeros_i
        return 0

    lax.fori_loop(0, NV, zero_body, 0, unroll=UNROLL)

    def issue_prefetch(row):
        pltpu.async_copy(xp_hbm.at[row], key_s.at[pl.ds(0, N)], dma_sem)
        pltpu.async_copy(yp_hbm.at[row], key_s.at[pl.ds(N, M)], dma_sem)
        pltpu.async_copy(x_hbm.at[row], val_s.at[pl.ds(0, N)], dma_sem)
        pltpu.async_copy(y_hbm.at[row], val_s.at[pl.ds(N, M)], dma_sem)

    issue_prefetch(base)

    def row_body(r, _):
        row = base + r
        pltpu.make_async_copy(xp_hbm.at[row], key_s.at[pl.ds(0, N)], dma_sem).wait()
        pltpu.make_async_copy(yp_hbm.at[row], key_s.at[pl.ds(N, M)], dma_sem).wait()
        pltpu.make_async_copy(x_hbm.at[row], val_s.at[pl.ds(0, N)], dma_sem).wait()
        pltpu.make_async_copy(y_hbm.at[row], val_s.at[pl.ds(N, M)], dma_sem).wait()

        scale_vecs = [None]

        for p in range(NPASS):
            shift = 8 * p
            if p == 0:
                src_k, src_v = key_s, val_s
            elif p % 2 == 0:
                src_k, src_v = key_b, val_b
            else:
                src_k, src_v = key_a, val_a
            dst_k, dst_v = (key_a, val_a) if p % 2 == 0 else (key_b, val_b)

            if p == 0:
                def hist0_body(i, carry):
                    ax, ay = carry
                    k = src_k[pl.ds(i * 16, 16)]
                    ki = plsc.bitcast(k, jnp.int32)
                    d = ki & (NDIG - 1)
                    plsc.addupdate_scatter(hist, [(lanes << 8) | d], ones_i)
                    v = src_v[pl.ds(i * 16, 16)]
                    is_x = i < NVX
                    ax = ax + jnp.where(is_x, v, zeros_f)
                    ay = ay + jnp.where(is_x, zeros_f, v)
                    return ax, ay

                ax, ay = lax.fori_loop(0, NV, hist0_body, (zeros_f, zeros_f),
                                       unroll=UNROLL)
                inv_sx = ones_f / (ones_f * jnp.sum(ax))
                neg_inv_sy = (-ones_f) / (ones_f * jnp.sum(ay))
                scale_vecs[0] = (inv_sx, neg_inv_sy)
            else:
                def hist_body(i, _, shift=shift, src_k=src_k):
                    k = src_k[pl.ds(i * 16, 16)]
                    ki = plsc.bitcast(k, jnp.int32)
                    d = lax.shift_right_logical(ki, shift) & (NDIG - 1)
                    plsc.addupdate_scatter(hist, [(lanes << 8) | d], ones_i)
                    return 0

                lax.fori_loop(0, NV, hist_body, 0, unroll=UNROLL)

            def off_group(j, carry):
                acc = zeros_i
                es = []
                for l in range(16):
                    sl = pl.ds(l * NDIG + j * 16, 16)
                    hv = hist[sl]
                    hist[sl] = zeros_i
                    es.append(acc)
                    acc = acc + hv
                incl = plsc.cumsum(acc)
                base_vec = carry + incl - acc
                for l in range(16):
                    off[pl.ds(l * NDIG + j * 16, 16)] = base_vec + es[l]
                return carry + incl[15]

            lax.fori_loop(0, NDIG // 16, off_group, jnp.int32(0))

            if p == 0:
                inv_sx, neg_inv_sy = scale_vecs[0]

                def perm0_body(i, _):
                    sl = pl.ds(i * 16, 16)
                    k = src_k[sl]
                    v = src_v[sl]
                    ki = plsc.bitcast(k, jnp.int32)
                    d = ki & (NDIG - 1)
                    taddr = (lanes << 8) | d
                    g = plsc.load_gather(off, [taddr])
                    plsc.addupdate_scatter(off, [taddr], ones_i)
                    a = ((g & (NV - 1)) << 4) | lax.shift_right_logical(g, 8)
                    plsc.store_scatter(dst_k, [a], k)
                    v = v * jnp.where(i < NVX, inv_sx, neg_inv_sy)
                    plsc.store_scatter(dst_v, [a], v)
                    return 0

                lax.fori_loop(0, NV, perm0_body, 0, unroll=UNROLL)
                @pl.when(r + 1 < RPW)
                def _():
                    issue_prefetch(row + 1)
            else:
                def perm_body(i, _, shift=shift, src_k=src_k, src_v=src_v,
                              dst_k=dst_k, dst_v=dst_v):
                    sl = pl.ds(i * 16, 16)
                    k = src_k[sl]
                    v = src_v[sl]
                    ki = plsc.bitcast(k, jnp.int32)
                    d = lax.shift_right_logical(ki, shift) & (NDIG - 1)
                    taddr = (lanes << 8) | d
                    g = plsc.load_gather(off, [taddr])
                    plsc.addupdate_scatter(off, [taddr], ones_i)
                    a = ((g & (NV - 1)) << 4) | lax.shift_right_logical(g, 8)
                    plsc.store_scatter(dst_k, [a], k)
                    plsc.store_scatter(dst_v, [a], v)
                    return 0

                lax.fori_loop(0, NV, perm_body, 0, unroll=UNROLL)

        def tot_body(i, acc):
            return acc + val_b[pl.ds(i * 16, 16)]

        t = lax.fori_loop(0, NV, tot_body, zeros_f, unroll=UNROLL)
        lane_base = plsc.cumsum(t) - t

        def loss_body(i, carry):
            k, acc, lsum = carry
            nk = key_b[pl.ds((i + 1) * 16, 16)]
            acc = acc + val_b[pl.ds(i * 16, 16)]
            lsum = lsum + (nk - k) * jnp.abs(lane_base + acc)
            return nk, acc, lsum

        k_end, acc, lsum = lax.fori_loop(
            0, NV - 1, loss_body,
            (key_b[pl.ds(0, 16)], zeros_f, zeros_f), unroll=UNROLL)
        acc = acc + val_b[pl.ds((NV - 1) * 16, 16)]
        d_end = lane_base + acc
        nxt_start = plsc.load_gather(key_b, [jnp.minimum(lanes + 1, 15)])
        seam = jnp.where(lanes < 15, (nxt_start - k_end) * jnp.abs(d_end), 0.0)
        loss = jnp.sum(lsum) + jnp.sum(seam)
        plsc.store_scatter(loss_buf, [ones_i * r], zeros_f + loss,
                           mask=lanes == 0)
        return 0

    lax.fori_loop(0, RPW, row_body, 0)
    pltpu.sync_copy(loss_buf, out_hbm.at[pl.ds(base, RPW)])


@jax.jit
def kernel(x, y, x_pos, y_pos):
    mesh = plsc.VectorSubcoreMesh(core_axis_name="c", subcore_axis_name="s",
                                  num_cores=NC, num_subcores=NS)
    f = pl.kernel(
        _sc_body,
        out_type=jax.ShapeDtypeStruct((B,), jnp.float32),
        mesh=mesh,
        scratch_types=[
            pltpu.VMEM((W,), jnp.float32),
            pltpu.VMEM((W,), jnp.float32),
            pltpu.VMEM((W,), jnp.float32),
            pltpu.VMEM((W,), jnp.float32),
            pltpu.VMEM((W,), jnp.float32),
            pltpu.VMEM((W,), jnp.float32),
            pltpu.VMEM((W,), jnp.int32),
            pltpu.VMEM((W,), jnp.int32),
            pltpu.VMEM((RPW,), jnp.float32),
            pltpu.SemaphoreType.DMA,
        ],
        compiler_params=pltpu.CompilerParams(needs_layout_passes=False),
    )
    return f(x, y, x_pos, y_pos)
